# baseline (speedup 1.0000x reference)
import functools

import jax
import jax.numpy as jnp
from jax import lax
from jax.experimental import pallas as pl
from jax.experimental.pallas import tpu as pltpu
from jax.experimental.pallas import tpu_sc as plsc

N = 10000
E = 320000
D = 128
EPS = 1e-5

_NC = 2
_NS = 16
_NW = _NC * _NS
_EPW = E // _NW
_KE = 80
_NIT = _EPW // _KE
_RPT = 632
_NPAD = _NS * _RPT

_mesh = plsc.VectorSubcoreMesh(core_axis_name="c", subcore_axis_name="s")


_NB = 4


def _segsum_body(h_hbm, row_hbm, col_hbm, zeros_hbm, out_hbm,
                 idx_r, idx_c, idx_s, rows_v, acc, sem_i, sem_g, sem_s):
    c = lax.axis_index("c")
    s = lax.axis_index("s")
    w = c * _NS + s
    rbase = pl.multiple_of(s * _RPT, 8)
    ebase = w * _EPW

    def issue_idx(chunk, b):
        base = pl.multiple_of(ebase + chunk * _KE, 8)
        pltpu.async_copy(row_hbm.at[pl.ds(base, _KE)], idx_r.at[b], sem_i.at[b])
        pltpu.async_copy(col_hbm.at[pl.ds(base, _KE)], idx_c.at[b], sem_i.at[b])

    def wait_idx(b):
        pltpu.make_async_copy(row_hbm.at[pl.ds(0, _KE)], idx_r.at[b],
                              sem_i.at[b]).wait()
        pltpu.make_async_copy(col_hbm.at[pl.ds(0, _KE)], idx_c.at[b],
                              sem_i.at[b]).wait()

    def wait_scatter(b):
        pltpu.make_async_copy(rows_v.at[b], acc.at[idx_s.at[b]],
                              sem_s.at[b]).wait()

    pltpu.sync_copy(zeros_hbm.at[pl.ds(rbase, _RPT)],
                    acc.at[pl.ds(rbase, _RPT)])
    plsc.subcore_barrier()

    issue_idx(0, 0)
    issue_idx(1, 1)

    @pl.loop(0, _NIT)
    def _(i):
        b = lax.rem(i, _NB)
        wait_idx(b)

        @pl.when(i >= _NB)
        def _():
            wait_scatter(b)

        pltpu.async_copy(h_hbm.at[idx_c.at[b]], rows_v.at[b], sem_g.at[b])

        @pl.when(i + 2 < _NIT)
        def _():
            issue_idx(i + 2, lax.rem(i + 2, _NB))

        @pl.when(i >= 1)
        def _():
            b1 = lax.rem(i + _NB - 1, _NB)
            pltpu.make_async_copy(h_hbm.at[idx_c.at[b1]], rows_v.at[b1],
                                  sem_g.at[b1]).wait()
            for k in range(_KE // 16):
                sl = pl.ds(16 * k, 16)
                idx_s[b1, sl] = idx_r[b1, sl]
            pltpu.async_copy(rows_v.at[b1], acc.at[idx_s.at[b1]],
                             sem_s.at[b1], add=True)

    bl = (_NIT - 1) % _NB
    pltpu.make_async_copy(h_hbm.at[idx_c.at[bl]], rows_v.at[bl],
                          sem_g.at[bl]).wait()
    for k in range(_KE // 16):
        sl = pl.ds(16 * k, 16)
        idx_s[bl, sl] = idx_r[bl, sl]
    pltpu.async_copy(rows_v.at[bl], acc.at[idx_s.at[bl]], sem_s.at[bl],
                     add=True)
    for b in range(_NB):
        wait_scatter(b)

    plsc.subcore_barrier()
    pltpu.sync_copy(acc.at[pl.ds(rbase, _RPT)],
                    out_hbm.at[c, pl.ds(rbase, _RPT)])


_segsum = pl.kernel(
    _segsum_body,
    out_type=jax.ShapeDtypeStruct((_NC, _NPAD, D), jnp.float32),
    mesh=_mesh,
    scratch_types=[
        pltpu.VMEM((_NB, _KE), jnp.int32),
        pltpu.VMEM((_NB, _KE), jnp.int32),
        pltpu.VMEM((_NB, _KE), jnp.int32),
        pltpu.VMEM((_NB, _KE, D), jnp.float32),
        pltpu.VMEM_SHARED((_NPAD, D), jnp.float32),
        pltpu.SemaphoreType.DMA((_NB,)),
        pltpu.SemaphoreType.DMA((_NB,)),
        pltpu.SemaphoreType.DMA((_NB,)),
    ],
)


_EH = E // 2
_EPWH = _EH // _NW
_KEH = 40
_NITH = _EPWH // _KEH


def _make_edgesum_half(e0):
    def body(p_hbm, q_hbm, row_hbm, col_hbm, iota_hbm, out_hbm,
             idx_r, idx_c, ids, buf_a, buf_b, stage,
             sem_i, sem_p, sem_q, sem_cp, sem_sa, sem_o):
        c = lax.axis_index("c")
        s = lax.axis_index("s")
        w = c * _NS + s
        ebase = e0 + w * _EPWH

        for b in range(_NB):
            sbase = pl.multiple_of((s * _NB + b) * _KEH, 8)
            pltpu.sync_copy(iota_hbm.at[pl.ds(sbase, _KEH)], ids.at[b])

        def issue_idx(chunk, b):
            base = pl.multiple_of(ebase + chunk * _KEH, 8)
            pltpu.async_copy(row_hbm.at[pl.ds(base, _KEH)], idx_r.at[b],
                             sem_i.at[b])
            pltpu.async_copy(col_hbm.at[pl.ds(base, _KEH)], idx_c.at[b],
                             sem_i.at[b])

        def wait_idx(b):
            pltpu.make_async_copy(row_hbm.at[pl.ds(0, _KEH)], idx_r.at[b],
                                  sem_i.at[b]).wait()
            pltpu.make_async_copy(col_hbm.at[pl.ds(0, _KEH)], idx_c.at[b],
                                  sem_i.at[b]).wait()

        def slot(b):
            return pl.multiple_of((s * _NB + b) * _KEH, 8)

        def stage_copy(b1):
            pltpu.make_async_copy(p_hbm.at[idx_r.at[b1]], buf_a.at[b1],
                                  sem_p.at[b1]).wait()
            pltpu.make_async_copy(q_hbm.at[idx_c.at[b1]], buf_b.at[b1],
                                  sem_q.at[b1]).wait()
            pltpu.async_copy(buf_a.at[b1], stage.at[pl.ds(slot(b1), _KEH)],
                             sem_cp.at[b1])

        def stage_add(b2):
            pltpu.make_async_copy(buf_a.at[b2],
                                  stage.at[pl.ds(slot(b2), _KEH)],
                                  sem_cp.at[b2]).wait()
            pltpu.async_copy(buf_b.at[b2], stage.at[ids.at[b2]],
                             sem_sa.at[b2], add=True)

        def stage_out(chunk, b3):
            pltpu.make_async_copy(buf_b.at[b3], stage.at[ids.at[b3]],
                                  sem_sa.at[b3]).wait()
            base = pl.multiple_of(ebase - e0 + chunk * _KEH, 8)
            pltpu.async_copy(stage.at[pl.ds(slot(b3), _KEH)],
                             out_hbm.at[pl.ds(base, _KEH)], sem_o.at[b3])

        def wait_out(b):
            pltpu.make_async_copy(stage.at[pl.ds(slot(b), _KEH)],
                                  out_hbm.at[pl.ds(0, _KEH)],
                                  sem_o.at[b]).wait()

        issue_idx(0, 0)
        issue_idx(1, 1)

        @pl.loop(0, _NITH)
        def _(i):
            b = lax.rem(i, _NB)
            wait_idx(b)

            @pl.when(i >= _NB)
            def _():
                wait_out(b)

            pltpu.async_copy(p_hbm.at[idx_r.at[b]], buf_a.at[b], sem_p.at[b])
            pltpu.async_copy(q_hbm.at[idx_c.at[b]], buf_b.at[b], sem_q.at[b])

            @pl.when(i >= 1)
            def _():
                stage_copy(lax.rem(i + _NB - 1, _NB))

            @pl.when(i + 2 < _NITH)
            def _():
                issue_idx(i + 2, lax.rem(i + 2, _NB))

            @pl.when(i >= 2)
            def _():
                stage_add(lax.rem(i + _NB - 2, _NB))

            @pl.when(i >= 3)
            def _():
                stage_out(i - 3, lax.rem(i + _NB - 3, _NB))

        stage_copy((_NITH - 1) % _NB)
        stage_add((_NITH - 2) % _NB)
        stage_add((_NITH - 1) % _NB)
        stage_out(_NITH - 3, (_NITH - 3) % _NB)
        stage_out(_NITH - 2, (_NITH - 2) % _NB)
        stage_out(_NITH - 1, (_NITH - 1) % _NB)
        for b in range(_NB):
            wait_out(b)

    return pl.kernel(
        body,
        out_type=jax.ShapeDtypeStruct((_EH, D), jnp.float32),
        mesh=_mesh,
        scratch_types=[
            pltpu.VMEM((_NB, _KEH), jnp.int32),
            pltpu.VMEM((_NB, _KEH), jnp.int32),
            pltpu.VMEM((_NB, _KEH), jnp.int32),
            pltpu.VMEM((_NB, _KEH, D), jnp.float32),
            pltpu.VMEM((_NB, _KEH, D), jnp.float32),
            pltpu.VMEM_SHARED((_NS * _NB * _KEH, D), jnp.float32),
            pltpu.SemaphoreType.DMA((_NB,)),
            pltpu.SemaphoreType.DMA((_NB,)),
            pltpu.SemaphoreType.DMA((_NB,)),
            pltpu.SemaphoreType.DMA((_NB,)),
            pltpu.SemaphoreType.DMA((_NB,)),
            pltpu.SemaphoreType.DMA((_NB,)),
        ],
    )


_edgesum_a = _make_edgesum_half(0)
_edgesum_b = _make_edgesum_half(_EH)


def _ln_rows(y, g, b):
    m = jnp.mean(y, axis=-1, keepdims=True)
    v = jnp.mean((y - m) ** 2, axis=-1, keepdims=True)
    return (y - m) * lax.rsqrt(v + EPS) * g + b


def _proj_body(x_ref, w_ref, b_ref, o_ref):
    o_ref[...] = (
        jnp.dot(x_ref[...], w_ref[...], preferred_element_type=jnp.float32)
        + b_ref[...]
    )


def _node_body(h_ref, p_ref, wh_ref, wa_ref, b_ref, g_ref, beta_ref, o_ref):
    h = h_ref[...]
    agg = p_ref[0] + p_ref[1]
    y = (
        jnp.dot(h, wh_ref[...], preferred_element_type=jnp.float32)
        + jnp.dot(agg, wa_ref[...], preferred_element_type=jnp.float32)
        + b_ref[...]
    )
    y = jnp.maximum(y, 0.0)
    o_ref[...] = _ln_rows(y, g_ref[...], beta_ref[...])


def _node_pq_body(h_ref, p_ref, wh_ref, wa_ref, b_ref, g_ref, beta_ref,
                  w1_ref, w2_ref, be_ref, o_ref, pe_ref, qe_ref):
    h = h_ref[...]
    agg = p_ref[0] + p_ref[1]
    y = (
        jnp.dot(h, wh_ref[...], preferred_element_type=jnp.float32)
        + jnp.dot(agg, wa_ref[...], preferred_element_type=jnp.float32)
        + b_ref[...]
    )
    y = jnp.maximum(y, 0.0)
    o_ref[...] = _ln_rows(y, g_ref[...], beta_ref[...])
    pe_ref[...] = (
        jnp.dot(h, w1_ref[...], preferred_element_type=jnp.float32)
        + be_ref[...]
    )
    qe_ref[...] = jnp.dot(h, w2_ref[...], preferred_element_type=jnp.float32)


def _eln_body(y_ref, g_ref, beta_ref, o_ref):
    y = jnp.maximum(y_ref[...], 0.0)
    o_ref[...] = _ln_rows(y, g_ref[...], beta_ref[...])


_BN = 2000
_BE = 8000

_full = lambda shape: pl.BlockSpec(shape, lambda i: (0,) * len(shape))
_rows = lambda bs: pl.BlockSpec((bs, D), lambda i: (i, 0))


def _proj(x, w, b):
    return pl.pallas_call(
        _proj_body,
        out_shape=jax.ShapeDtypeStruct((N, D), jnp.float32),
        grid=(N // _BN,),
        in_specs=[_rows(_BN), _full((D, D)), _full((1, D))],
        out_specs=_rows(_BN),
    )(x, w, b)


def _node_update(h, parts, wh, wa, b, g, beta):
    return pl.pallas_call(
        _node_body,
        out_shape=jax.ShapeDtypeStruct((N, D), jnp.float32),
        grid=(N // _BN,),
        in_specs=[
            _rows(_BN),
            pl.BlockSpec((_NC, _BN, D), lambda i: (0, i, 0)),
            _full((D, D)), _full((D, D)),
            _full((1, D)), _full((1, D)), _full((1, D)),
        ],
        out_specs=_rows(_BN),
    )(h, parts, wh, wa, b, g, beta)


def _node_update_pq(h, parts, wh, wa, b, g, beta, w1, w2, be_):
    return pl.pallas_call(
        _node_pq_body,
        out_shape=(
            jax.ShapeDtypeStruct((N, D), jnp.float32),
            jax.ShapeDtypeStruct((N, D), jnp.float32),
            jax.ShapeDtypeStruct((N, D), jnp.float32),
        ),
        grid=(N // _BN,),
        in_specs=[
            _rows(_BN),
            pl.BlockSpec((_NC, _BN, D), lambda i: (0, i, 0)),
            _full((D, D)), _full((D, D)),
            _full((1, D)), _full((1, D)), _full((1, D)),
            _full((D, D)), _full((D, D)), _full((1, D)),
        ],
        out_specs=(_rows(_BN), _rows(_BN), _rows(_BN)),
    )(h, parts, wh, wa, b, g, beta, w1, w2, be_)


def _eln_body2(y_ref, g_ref, beta_ref, alias_ref, o_ref):
    del alias_ref
    y = jnp.maximum(y_ref[...], 0.0)
    o_ref[...] = _ln_rows(y, g_ref[...], beta_ref[...])


def _edge_ln_a(y, g, beta):
    return pl.pallas_call(
        _eln_body,
        out_shape=jax.ShapeDtypeStruct((E, D), jnp.float32),
        grid=(_EH // _BE,),
        in_specs=[_rows(_BE), _full((1, D)), _full((1, D))],
        out_specs=_rows(_BE),
    )(y, g, beta)


def _edge_ln_b(y, g, beta, e_a):
    nblk = _EH // _BE
    return pl.pallas_call(
        _eln_body2,
        out_shape=jax.ShapeDtypeStruct((E, D), jnp.float32),
        grid=(nblk,),
        in_specs=[
            _rows(_BE), _full((1, D)), _full((1, D)),
            pl.BlockSpec(memory_space=pl.ANY),
        ],
        out_specs=pl.BlockSpec((_BE, D), lambda i: (i + nblk, 0)),
        input_output_aliases={3: 0},
    )(y, g, beta, e_a)


def kernel(x, edge_index, edge_attr, W_node, b_node, W_edge, b_edge,
           Wn, bn, gn, betan, We, be, ge, betae):
    del edge_attr, W_edge, b_edge
    row = edge_index[0].astype(jnp.int32)
    col = edge_index[1].astype(jnp.int32)
    zeros = jnp.zeros((_NPAD, D), jnp.float32)

    h = _proj(x, W_node, b_node.reshape(1, D))
    e = None
    for l in range(3):
        parts = _segsum(h, row, col, zeros)[:, :N, :]
        if l < 2:
            h = _node_update(
                h, parts, Wn[l, :D], Wn[l, D:],
                bn[l].reshape(1, D), gn[l].reshape(1, D),
                betan[l].reshape(1, D),
            )
        else:
            h, p, q = _node_update_pq(
                h, parts, Wn[l, :D], Wn[l, D:],
                bn[l].reshape(1, D), gn[l].reshape(1, D),
                betan[l].reshape(1, D),
                We[l, :D], We[l, D:], be[l].reshape(1, D),
            )
            iota = jnp.arange(_NS * _NB * _KEH, dtype=jnp.int32)
            ge_, betae_ = ge[l].reshape(1, D), betae[l].reshape(1, D)
            epre_a = _edgesum_a(p, q, row, col, iota)
            epre_b = _edgesum_b(p, q, row, col, iota)
            e_a = _edge_ln_a(epre_a, ge_, betae_)
            e = _edge_ln_b(epre_b, ge_, betae_, e_a)
    return (h, e)


# speedup vs baseline: 1.0530x; 1.0530x over previous
import functools

import jax
import jax.numpy as jnp
from jax import lax
from jax.experimental import pallas as pl
from jax.experimental.pallas import tpu as pltpu
from jax.experimental.pallas import tpu_sc as plsc

N = 10000
E = 320000
D = 128
EPS = 1e-5

_NC = 2
_NS = 16
_NW = _NC * _NS
_EPW = E // _NW
_KE = 80
_NIT = _EPW // _KE
_RPT = 632
_NPAD = _NS * _RPT

_mesh = plsc.VectorSubcoreMesh(core_axis_name="c", subcore_axis_name="s")


_NB = 4


def _segsum_body(h_hbm, row_hbm, col_hbm, zeros_hbm, out_hbm,
                 idx_r, idx_c, idx_s, rows_v, acc, sem_i, sem_g, sem_s):
    c = lax.axis_index("c")
    s = lax.axis_index("s")
    w = c * _NS + s
    rbase = pl.multiple_of(s * _RPT, 8)
    ebase = w * _EPW

    def issue_idx(chunk, b):
        base = pl.multiple_of(ebase + chunk * _KE, 8)
        pltpu.async_copy(row_hbm.at[pl.ds(base, _KE)], idx_r.at[b], sem_i.at[b])
        pltpu.async_copy(col_hbm.at[pl.ds(base, _KE)], idx_c.at[b], sem_i.at[b])

    def wait_idx(b):
        pltpu.make_async_copy(row_hbm.at[pl.ds(0, _KE)], idx_r.at[b],
                              sem_i.at[b]).wait()
        pltpu.make_async_copy(col_hbm.at[pl.ds(0, _KE)], idx_c.at[b],
                              sem_i.at[b]).wait()

    def wait_scatter(b):
        pltpu.make_async_copy(rows_v.at[b], acc.at[idx_s.at[b]],
                              sem_s.at[b]).wait()

    pltpu.sync_copy(zeros_hbm.at[pl.ds(rbase, _RPT)],
                    acc.at[pl.ds(rbase, _RPT)])
    plsc.subcore_barrier()

    issue_idx(0, 0)
    issue_idx(1, 1)

    @pl.loop(0, _NIT)
    def _(i):
        b = lax.rem(i, _NB)
        wait_idx(b)

        @pl.when(i >= _NB)
        def _():
            wait_scatter(b)

        pltpu.async_copy(h_hbm.at[idx_c.at[b]], rows_v.at[b], sem_g.at[b])

        @pl.when(i + 2 < _NIT)
        def _():
            issue_idx(i + 2, lax.rem(i + 2, _NB))

        @pl.when(i >= 1)
        def _():
            b1 = lax.rem(i + _NB - 1, _NB)
            pltpu.make_async_copy(h_hbm.at[idx_c.at[b1]], rows_v.at[b1],
                                  sem_g.at[b1]).wait()
            for k in range(_KE // 16):
                sl = pl.ds(16 * k, 16)
                idx_s[b1, sl] = idx_r[b1, sl]
            pltpu.async_copy(rows_v.at[b1], acc.at[idx_s.at[b1]],
                             sem_s.at[b1], add=True)

    bl = (_NIT - 1) % _NB
    pltpu.make_async_copy(h_hbm.at[idx_c.at[bl]], rows_v.at[bl],
                          sem_g.at[bl]).wait()
    for k in range(_KE // 16):
        sl = pl.ds(16 * k, 16)
        idx_s[bl, sl] = idx_r[bl, sl]
    pltpu.async_copy(rows_v.at[bl], acc.at[idx_s.at[bl]], sem_s.at[bl],
                     add=True)
    for b in range(_NB):
        wait_scatter(b)

    plsc.subcore_barrier()
    pltpu.sync_copy(acc.at[pl.ds(rbase, _RPT)],
                    out_hbm.at[c, pl.ds(rbase, _RPT)])


_segsum = pl.kernel(
    _segsum_body,
    out_type=jax.ShapeDtypeStruct((_NC, _NPAD, D), jnp.float32),
    mesh=_mesh,
    scratch_types=[
        pltpu.VMEM((_NB, _KE), jnp.int32),
        pltpu.VMEM((_NB, _KE), jnp.int32),
        pltpu.VMEM((_NB, _KE), jnp.int32),
        pltpu.VMEM((_NB, _KE, D), jnp.float32),
        pltpu.VMEM_SHARED((_NPAD, D), jnp.float32),
        pltpu.SemaphoreType.DMA((_NB,)),
        pltpu.SemaphoreType.DMA((_NB,)),
        pltpu.SemaphoreType.DMA((_NB,)),
    ],
)


def _edgesum_body(p_hbm, q_hbm, row_hbm, col_hbm, out_hbm,
                  idx_r, idx_c, ids, buf_a, buf_b, stage,
                  sem_i, sem_p, sem_q, sem_cp, sem_sa, sem_o):
    c = lax.axis_index("c")
    s = lax.axis_index("s")
    w = c * _NS + s
    ebase = w * _EPW

    lane = jax.lax.iota(jnp.int32, 16)
    for b in range(_NB):
        for k in range(_KE // 16):
            ids[b, pl.ds(16 * k, 16)] = lane + (s * _NB + b) * _KE + 16 * k

    def issue_idx(chunk, b):
        base = pl.multiple_of(ebase + chunk * _KE, 8)
        pltpu.async_copy(row_hbm.at[pl.ds(base, _KE)], idx_r.at[b], sem_i.at[b])
        pltpu.async_copy(col_hbm.at[pl.ds(base, _KE)], idx_c.at[b], sem_i.at[b])

    def wait_idx(b):
        pltpu.make_async_copy(row_hbm.at[pl.ds(0, _KE)], idx_r.at[b],
                              sem_i.at[b]).wait()
        pltpu.make_async_copy(col_hbm.at[pl.ds(0, _KE)], idx_c.at[b],
                              sem_i.at[b]).wait()

    def slot(b):
        return pl.multiple_of((s * _NB + b) * _KE, 8)

    def stage_copy(b1):
        pltpu.make_async_copy(p_hbm.at[idx_r.at[b1]], buf_a.at[b1],
                              sem_p.at[b1]).wait()
        pltpu.make_async_copy(q_hbm.at[idx_c.at[b1]], buf_b.at[b1],
                              sem_q.at[b1]).wait()
        pltpu.async_copy(buf_a.at[b1], stage.at[pl.ds(slot(b1), _KE)],
                         sem_cp.at[b1])

    def stage_add(b2):
        pltpu.make_async_copy(buf_a.at[b2], stage.at[pl.ds(slot(b2), _KE)],
                              sem_cp.at[b2]).wait()
        pltpu.async_copy(buf_b.at[b2], stage.at[ids.at[b2]], sem_sa.at[b2],
                         add=True)

    def stage_out(chunk, b3):
        pltpu.make_async_copy(buf_b.at[b3], stage.at[ids.at[b3]],
                              sem_sa.at[b3]).wait()
        base = pl.multiple_of(ebase + chunk * _KE, 8)
        pltpu.async_copy(stage.at[pl.ds(slot(b3), _KE)],
                         out_hbm.at[pl.ds(base, _KE)], sem_o.at[b3])

    def wait_out(b):
        pltpu.make_async_copy(stage.at[pl.ds(slot(b), _KE)],
                              out_hbm.at[pl.ds(0, _KE)], sem_o.at[b]).wait()

    issue_idx(0, 0)
    issue_idx(1, 1)

    @pl.loop(0, _NIT)
    def _(i):
        b = lax.rem(i, _NB)
        wait_idx(b)

        @pl.when(i >= _NB)
        def _():
            wait_out(b)

        pltpu.async_copy(p_hbm.at[idx_r.at[b]], buf_a.at[b], sem_p.at[b])
        pltpu.async_copy(q_hbm.at[idx_c.at[b]], buf_b.at[b], sem_q.at[b])

        @pl.when(i >= 1)
        def _():
            stage_copy(lax.rem(i + _NB - 1, _NB))

        @pl.when(i + 2 < _NIT)
        def _():
            issue_idx(i + 2, lax.rem(i + 2, _NB))

        @pl.when(i >= 2)
        def _():
            stage_add(lax.rem(i + _NB - 2, _NB))

        @pl.when(i >= 3)
        def _():
            stage_out(i - 3, lax.rem(i + _NB - 3, _NB))

    stage_copy((_NIT - 1) % _NB)
    stage_add((_NIT - 2) % _NB)
    stage_add((_NIT - 1) % _NB)
    stage_out(_NIT - 3, (_NIT - 3) % _NB)
    stage_out(_NIT - 2, (_NIT - 2) % _NB)
    stage_out(_NIT - 1, (_NIT - 1) % _NB)
    for b in range(_NB):
        wait_out(b)


_edgesum = pl.kernel(
    _edgesum_body,
    out_type=jax.ShapeDtypeStruct((E, D), jnp.float32),
    mesh=_mesh,
    scratch_types=[
        pltpu.VMEM((_NB, _KE), jnp.int32),
        pltpu.VMEM((_NB, _KE), jnp.int32),
        pltpu.VMEM((_NB, _KE), jnp.int32),
        pltpu.VMEM((_NB, _KE, D), jnp.float32),
        pltpu.VMEM((_NB, _KE, D), jnp.float32),
        pltpu.VMEM_SHARED((_NS * _NB * _KE, D), jnp.float32),
        pltpu.SemaphoreType.DMA((_NB,)),
        pltpu.SemaphoreType.DMA((_NB,)),
        pltpu.SemaphoreType.DMA((_NB,)),
        pltpu.SemaphoreType.DMA((_NB,)),
        pltpu.SemaphoreType.DMA((_NB,)),
        pltpu.SemaphoreType.DMA((_NB,)),
    ],
)


def _ln_rows(y, g, b):
    m = jnp.mean(y, axis=-1, keepdims=True)
    v = jnp.mean((y - m) ** 2, axis=-1, keepdims=True)
    return (y - m) * lax.rsqrt(v + EPS) * g + b


def _proj_body(x_ref, w_ref, b_ref, o_ref):
    o_ref[...] = (
        jnp.dot(x_ref[...], w_ref[...], preferred_element_type=jnp.float32)
        + b_ref[...]
    )


def _node_body(h_ref, p_ref, wh_ref, wa_ref, b_ref, g_ref, beta_ref, o_ref):
    h = h_ref[...]
    agg = p_ref[0] + p_ref[1]
    y = (
        jnp.dot(h, wh_ref[...], preferred_element_type=jnp.float32)
        + jnp.dot(agg, wa_ref[...], preferred_element_type=jnp.float32)
        + b_ref[...]
    )
    y = jnp.maximum(y, 0.0)
    o_ref[...] = _ln_rows(y, g_ref[...], beta_ref[...])


def _node_pq_body(h_ref, p_ref, wh_ref, wa_ref, b_ref, g_ref, beta_ref,
                  w1_ref, w2_ref, be_ref, o_ref, pe_ref, qe_ref):
    h = h_ref[...]
    agg = p_ref[0] + p_ref[1]
    y = (
        jnp.dot(h, wh_ref[...], preferred_element_type=jnp.float32)
        + jnp.dot(agg, wa_ref[...], preferred_element_type=jnp.float32)
        + b_ref[...]
    )
    y = jnp.maximum(y, 0.0)
    o_ref[...] = _ln_rows(y, g_ref[...], beta_ref[...])
    pe_ref[...] = (
        jnp.dot(h, w1_ref[...], preferred_element_type=jnp.float32)
        + be_ref[...]
    )
    qe_ref[...] = jnp.dot(h, w2_ref[...], preferred_element_type=jnp.float32)


def _eln_body(y_ref, g_ref, beta_ref, o_ref):
    y = jnp.maximum(y_ref[...], 0.0)
    o_ref[...] = _ln_rows(y, g_ref[...], beta_ref[...])


_BN = 2000
_BE = 8000

_full = lambda shape: pl.BlockSpec(shape, lambda i: (0,) * len(shape))
_rows = lambda bs: pl.BlockSpec((bs, D), lambda i: (i, 0))


def _proj(x, w, b):
    return pl.pallas_call(
        _proj_body,
        out_shape=jax.ShapeDtypeStruct((N, D), jnp.float32),
        grid=(N // _BN,),
        in_specs=[_rows(_BN), _full((D, D)), _full((1, D))],
        out_specs=_rows(_BN),
    )(x, w, b)


def _node_update(h, parts, wh, wa, b, g, beta):
    return pl.pallas_call(
        _node_body,
        out_shape=jax.ShapeDtypeStruct((N, D), jnp.float32),
        grid=(N // _BN,),
        in_specs=[
            _rows(_BN),
            pl.BlockSpec((_NC, _BN, D), lambda i: (0, i, 0)),
            _full((D, D)), _full((D, D)),
            _full((1, D)), _full((1, D)), _full((1, D)),
        ],
        out_specs=_rows(_BN),
    )(h, parts, wh, wa, b, g, beta)


def _node_update_pq(h, parts, wh, wa, b, g, beta, w1, w2, be_):
    return pl.pallas_call(
        _node_pq_body,
        out_shape=(
            jax.ShapeDtypeStruct((N, D), jnp.float32),
            jax.ShapeDtypeStruct((N, D), jnp.float32),
            jax.ShapeDtypeStruct((N, D), jnp.float32),
        ),
        grid=(N // _BN,),
        in_specs=[
            _rows(_BN),
            pl.BlockSpec((_NC, _BN, D), lambda i: (0, i, 0)),
            _full((D, D)), _full((D, D)),
            _full((1, D)), _full((1, D)), _full((1, D)),
            _full((D, D)), _full((D, D)), _full((1, D)),
        ],
        out_specs=(_rows(_BN), _rows(_BN), _rows(_BN)),
    )(h, parts, wh, wa, b, g, beta, w1, w2, be_)


def _edge_ln(y, g, beta):
    return pl.pallas_call(
        _eln_body,
        out_shape=jax.ShapeDtypeStruct((E, D), jnp.float32),
        grid=(E // _BE,),
        in_specs=[_rows(_BE), _full((1, D)), _full((1, D))],
        out_specs=_rows(_BE),
    )(y, g, beta)


def kernel(x, edge_index, edge_attr, W_node, b_node, W_edge, b_edge,
           Wn, bn, gn, betan, We, be, ge, betae):
    del edge_attr, W_edge, b_edge
    row = edge_index[0].astype(jnp.int32)
    col = edge_index[1].astype(jnp.int32)
    zeros = jnp.zeros((_NPAD, D), jnp.float32)

    h = _proj(x, W_node, b_node.reshape(1, D))
    e = None
    for l in range(3):
        parts = _segsum(h, row, col, zeros)[:, :N, :]
        if l < 2:
            h = _node_update(
                h, parts, Wn[l, :D], Wn[l, D:],
                bn[l].reshape(1, D), gn[l].reshape(1, D),
                betan[l].reshape(1, D),
            )
        else:
            h, p, q = _node_update_pq(
                h, parts, Wn[l, :D], Wn[l, D:],
                bn[l].reshape(1, D), gn[l].reshape(1, D),
                betan[l].reshape(1, D),
                We[l, :D], We[l, D:], be[l].reshape(1, D),
            )
            epre = _edgesum(p, q, row, col)
            e = _edge_ln(epre, ge[l].reshape(1, D), betae[l].reshape(1, D))
    return (h, e)
